# baseline (speedup 1.0000x reference)
import functools
import jax
import jax.numpy as jnp
from jax import lax
from jax.experimental import pallas as pl
from jax.experimental.pallas import tpu as pltpu
from jax.experimental.pallas import tpu_sc as plsc

NN = 10000
FF = 128
EE = 320000

NCORE = 2
NSUB = 16
NWORK = NCORE * NSUB
KCH = 128
CCH = 80
EPAD = NWORK * CCH * KCH
NCHUNK = EPAD // KCH
CA = 160
CB = 2 * CCH - CA
RPT = 632
ACC_ROWS = NN + 8
RLAST = ACC_ROWS - (NSUB - 1) * RPT

BLK = 2000

_mesh = plsc.VectorSubcoreMesh(core_axis_name="c", subcore_axis_name="s")


def _zero_acc(zeros_hbm, acc_sh, s):
    def fill(nrows):
        for r in range(nrows // KCH):
            pltpu.sync_copy(zeros_hbm,
                            acc_sh.at[pl.ds(s * RPT + r * KCH, KCH)])
        rem = nrows % KCH
        pltpu.sync_copy(zeros_hbm.at[pl.ds(0, rem)],
                        acc_sh.at[pl.ds(s * RPT + (nrows - rem), rem)])

    @pl.when(s < NSUB - 1)
    def _():
        fill(RPT)

    @pl.when(s == NSUB - 1)
    def _():
        fill(RLAST)


def _copy_out(acc_sh, out_hbm, c, s):
    @pl.when(s < NSUB - 1)
    def _():
        pltpu.sync_copy(acc_sh.at[pl.ds(s * RPT, RPT)],
                        out_hbm.at[c, pl.ds(s * RPT, RPT)])

    @pl.when(s == NSUB - 1)
    def _():
        pltpu.sync_copy(acc_sh.at[pl.ds((NSUB - 1) * RPT, RLAST)],
                        out_hbm.at[c, pl.ds((NSUB - 1) * RPT, RLAST)])


def _unpack_chunk(pk2_v, sidx2_v, didx2_v, b):
    for kk in range(KCH // 16):
        v = pk2_v[b, 0, pl.ds(kk * 16, 16)]
        sidx2_v[b, pl.ds(kk * 16, 16)] = v & jnp.int32(0xFFFF)
        didx2_v[b, pl.ds(kk * 16, 16)] = lax.shift_right_logical(v, 16)


@functools.partial(
    pl.kernel,
    mesh=_mesh,
    out_type=jax.ShapeDtypeStruct((NCORE, ACC_ROWS, FF), jnp.float32),
    scratch_types=[
        pltpu.VMEM((CCH, KCH), jnp.int32),
        pltpu.VMEM((KCH, FF), jnp.float32),
        pltpu.VMEM_SHARED((ACC_ROWS, FF), jnp.float32),
    ],
)
def _sc_deg(pk_hbm, zeros_hbm, ones_hbm, out_hbm, pidx_v, ones_v, deg_sh):
    c = lax.axis_index("c")
    s = lax.axis_index("s")
    w = s * NCORE + c
    _zero_acc(zeros_hbm, deg_sh, s)
    pltpu.sync_copy(pk_hbm.at[pl.ds(w * CCH, CCH)], pidx_v)
    pltpu.sync_copy(ones_hbm, ones_v)

    def unp(r, carry):
        for kk in range(KCH // 16):
            v = pidx_v[r, pl.ds(kk * 16, 16)]
            pidx_v[r, pl.ds(kk * 16, 16)] = lax.shift_right_logical(v, 16)
        return carry

    lax.fori_loop(0, CCH, unp, 0)
    plsc.subcore_barrier()

    def body(i, carry):
        pltpu.sync_copy(ones_v, deg_sh.at[pidx_v.at[i]], add=True)
        return carry

    lax.fori_loop(0, CCH, body, 0)
    plsc.subcore_barrier()
    _copy_out(deg_sh, out_hbm, c, s)


@functools.partial(
    pl.kernel,
    mesh=_mesh,
    out_type=jax.ShapeDtypeStruct((NCORE, ACC_ROWS, FF), jnp.float32),
    scratch_types=[
        pltpu.VMEM((3, 1, KCH), jnp.int32),
        pltpu.VMEM((3, KCH), jnp.int32),
        pltpu.VMEM((3, KCH), jnp.int32),
        pltpu.VMEM((KCH, FF), jnp.float32),
        pltpu.VMEM((KCH, FF), jnp.float32),
        pltpu.VMEM((KCH, FF), jnp.float32),
        pltpu.VMEM_SHARED((ACC_ROWS, FF), jnp.float32),
        pltpu.SemaphoreType.DMA,
        pltpu.SemaphoreType.DMA,
        pltpu.SemaphoreType.DMA,
        pltpu.SemaphoreType.DMA,
        pltpu.SemaphoreType.DMA,
        pltpu.SemaphoreType.DMA,
    ],
)
def _sc_spmm(g_hbm, pk_hbm, zeros_hbm, out_hbm,
             pk2_v, sidx2_v, didx2_v, rows0_v, rows1_v, rows2_v, acc_sh,
             sg0, sg1, sg2, ss0, ss1, ss2):
    c = lax.axis_index("c")
    s = lax.axis_index("s")
    bufs = (rows0_v, rows1_v, rows2_v)
    gsems = (sg0, sg1, sg2)
    ssems = (ss0, ss1, ss2)
    with jax.named_scope("ph_zero"):
        _zero_acc(zeros_hbm, acc_sh, s)

    def fetch(off, q, b):
        pltpu.sync_copy(pk_hbm.at[pl.ds(off + q, 1)], pk2_v.at[pl.ds(b, 1)])
        _unpack_chunk(pk2_v, sidx2_v, didx2_v, b)

    def pre(off):
        fetch(off, 0, 0)
        pltpu.async_copy(g_hbm.at[sidx2_v.at[0]], bufs[0], gsems[0])
        fetch(off, 1, 1)
        pltpu.async_copy(g_hbm.at[sidx2_v.at[1]], bufs[1], gsems[1])

    def run(nch, off):
        def body(i, carry):
            for u in range(3):
                q = i * 3 + u
                b = u
                b2 = (u + 2) % 3

                @pl.when((q >= 1) & (q <= nch))
                def _():
                    pltpu.make_async_copy(g_hbm.at[pl.ds(0, KCH)],
                                          bufs[b2], ssems[b2]).wait()

                @pl.when(q + 2 < nch)
                def _():
                    fetch(off, q + 2, b2)
                    pltpu.async_copy(g_hbm.at[sidx2_v.at[b2]], bufs[b2],
                                     gsems[b2])

                @pl.when(q < nch)
                def _():
                    pltpu.make_async_copy(g_hbm.at[pl.ds(0, KCH)], bufs[b],
                                          gsems[b]).wait()
                    pltpu.async_copy(bufs[b], acc_sh.at[didx2_v.at[b]],
                                     ssems[b], add=True)
            return carry

        lax.fori_loop(0, (nch + 2) // 3, body, 0)
        if nch % 3 == 0:
            pltpu.make_async_copy(g_hbm.at[pl.ds(0, KCH)],
                                  bufs[(nch - 1) % 3],
                                  ssems[(nch - 1) % 3]).wait()

    with jax.named_scope("ph_pre"):
        @pl.when(c == 0)
        def _():
            pre(s * CA)

        if CB:
            @pl.when(c == 1)
            def _():
                pre(NSUB * CA + s * CB)

        plsc.subcore_barrier()

    with jax.named_scope("ph_loop"):
        @pl.when(c == 0)
        def _():
            run(CA, s * CA)

        if CB:
            @pl.when(c == 1)
            def _():
                run(CB, NSUB * CA + s * CB)

        plsc.subcore_barrier()

    with jax.named_scope("ph_out"):
        _copy_out(acc_sh, out_hbm, c, s)


def _prep_body(parts_ref, dinv_ref):
    deg = parts_ref[0, :NN, :1] + parts_ref[1, :NN, :1] + 1.0
    dinv_ref[...] = lax.rsqrt(deg)


def _tc_prep(deg_parts):
    return pl.pallas_call(
        _prep_body,
        out_shape=jax.ShapeDtypeStruct((NN, 1), jnp.float32),
    )(deg_parts)


def _mm1_body(x_ref, w_ref, dinv_ref, hlin_ref, g_ref):
    hl = jnp.dot(x_ref[...], w_ref[...],
                 preferred_element_type=jnp.float32,
                 precision=lax.Precision.HIGHEST)
    hlin_ref[...] = hl
    g_ref[...] = hl * dinv_ref[...]


def _tc_mm1(x, w, dinv):
    return pl.pallas_call(
        _mm1_body,
        grid=(NN // BLK,),
        in_specs=[
            pl.BlockSpec((BLK, FF), lambda i: (i, 0)),
            pl.BlockSpec((FF, FF), lambda i: (0, 0)),
            pl.BlockSpec((BLK, 1), lambda i: (i, 0)),
        ],
        out_specs=[
            pl.BlockSpec((BLK, FF), lambda i: (i, 0)),
            pl.BlockSpec((BLK, FF), lambda i: (i, 0)),
        ],
        out_shape=[
            jax.ShapeDtypeStruct((NN, FF), jnp.float32),
            jax.ShapeDtypeStruct((NN, FF), jnp.float32),
        ],
    )(x, w, dinv)


def _mid_body(acc_ref, hlin_ref, dinv_ref, b_ref, w_ref, hlinn_ref, gn_ref):
    dv = dinv_ref[...]
    a = acc_ref[0] + acc_ref[1]
    sfull = dv * a + (dv * dv) * hlin_ref[...] + b_ref[...]
    h = jnp.maximum(sfull, 0.0)
    hn = jnp.dot(h, w_ref[...],
                 preferred_element_type=jnp.float32,
                 precision=lax.Precision.HIGHEST)
    hlinn_ref[...] = hn
    gn_ref[...] = hn * dv


def _tc_mid(acc, hlin, dinv, b, w):
    return pl.pallas_call(
        _mid_body,
        grid=(NN // BLK,),
        in_specs=[
            pl.BlockSpec((NCORE, BLK, FF), lambda i: (0, i, 0)),
            pl.BlockSpec((BLK, FF), lambda i: (i, 0)),
            pl.BlockSpec((BLK, 1), lambda i: (i, 0)),
            pl.BlockSpec((1, FF), lambda i: (0, 0)),
            pl.BlockSpec((FF, FF), lambda i: (0, 0)),
        ],
        out_specs=[
            pl.BlockSpec((BLK, FF), lambda i: (i, 0)),
            pl.BlockSpec((BLK, FF), lambda i: (i, 0)),
        ],
        out_shape=[
            jax.ShapeDtypeStruct((NN, FF), jnp.float32),
            jax.ShapeDtypeStruct((NN, FF), jnp.float32),
        ],
    )(acc, hlin, dinv, b, w)


def _final_body(acc_ref, hlin_ref, dinv_ref, b_ref, out_ref):
    dv = dinv_ref[...]
    a = acc_ref[0] + acc_ref[1]
    out_ref[...] = dv * a + (dv * dv) * hlin_ref[...] + b_ref[...]


def _tc_final(acc, hlin, dinv, b):
    return pl.pallas_call(
        _final_body,
        grid=(NN // BLK,),
        in_specs=[
            pl.BlockSpec((NCORE, BLK, FF), lambda i: (0, i, 0)),
            pl.BlockSpec((BLK, FF), lambda i: (i, 0)),
            pl.BlockSpec((BLK, 1), lambda i: (i, 0)),
            pl.BlockSpec((1, FF), lambda i: (0, 0)),
        ],
        out_specs=pl.BlockSpec((BLK, FF), lambda i: (i, 0)),
        out_shape=jax.ShapeDtypeStruct((NN, FF), jnp.float32),
    )(acc, hlin, dinv, b)


def kernel(x, edge_index, W1, b1, W2, b2, W3, b3):
    src = edge_index[0]
    dst = edge_index[1]
    pad = EPAD - EE
    packed = jnp.left_shift(dst, 16) | src
    pkflat = jnp.concatenate(
        [packed, jnp.full((pad,), NN << 16, jnp.int32)])
    pk2d = pkflat.reshape(NCHUNK, KCH)
    pk3d = pkflat.reshape(NCHUNK, 1, KCH)

    zeros_rows = jnp.zeros((KCH, FF), jnp.float32)
    ones_rows = jnp.ones((KCH, FF), jnp.float32)

    deg_parts = _sc_deg(pk2d, zeros_rows, ones_rows)
    dinv = _tc_prep(deg_parts)

    hlin1, g1 = _tc_mm1(x, W1, dinv)
    acc1 = _sc_spmm(g1, pk3d, zeros_rows)
    hlin2, g2 = _tc_mid(acc1, hlin1, dinv, b1.reshape(1, FF), W2)
    acc2 = _sc_spmm(g2, pk3d, zeros_rows)
    hlin3, g3 = _tc_mid(acc2, hlin2, dinv, b2.reshape(1, FF), W3)
    acc3 = _sc_spmm(g3, pk3d, zeros_rows)
    return _tc_final(acc3, hlin3, dinv, b3.reshape(1, FF))


# speedup vs baseline: 1.3671x; 1.3671x over previous
import functools
import jax
import jax.numpy as jnp
from jax import lax
from jax.experimental import pallas as pl
from jax.experimental.pallas import tpu as pltpu
from jax.experimental.pallas import tpu_sc as plsc

NN = 10000
FF = 128
EE = 320000

NCORE = 2
NSUB = 16
NWORK = NCORE * NSUB
KCH = 128
CCH = 80
EPAD = NWORK * CCH * KCH
NCHUNK = EPAD // KCH
CA = 144
CB = 2 * CCH - CA
RPT = 632
ACC_ROWS = NN + 8
RLAST = ACC_ROWS - (NSUB - 1) * RPT

BLK = 2000

_mesh = plsc.VectorSubcoreMesh(core_axis_name="c", subcore_axis_name="s")


def _zero_acc(zeros_hbm, acc_sh, s):
    def fill(nrows):
        for r in range(nrows // KCH):
            pltpu.sync_copy(zeros_hbm,
                            acc_sh.at[pl.ds(s * RPT + r * KCH, KCH)])
        rem = nrows % KCH
        pltpu.sync_copy(zeros_hbm.at[pl.ds(0, rem)],
                        acc_sh.at[pl.ds(s * RPT + (nrows - rem), rem)])

    @pl.when(s < NSUB - 1)
    def _():
        fill(RPT)

    @pl.when(s == NSUB - 1)
    def _():
        fill(RLAST)


def _copy_out(acc_sh, out_hbm, c, s):
    @pl.when(s < NSUB - 1)
    def _():
        pltpu.sync_copy(acc_sh.at[pl.ds(s * RPT, RPT)],
                        out_hbm.at[c, pl.ds(s * RPT, RPT)])

    @pl.when(s == NSUB - 1)
    def _():
        pltpu.sync_copy(acc_sh.at[pl.ds((NSUB - 1) * RPT, RLAST)],
                        out_hbm.at[c, pl.ds((NSUB - 1) * RPT, RLAST)])


def _unpack_chunk(pk2_v, sidx2_v, didx2_v, b):
    for kk in range(KCH // 16):
        v = pk2_v[b, 0, pl.ds(kk * 16, 16)]
        sidx2_v[b, pl.ds(kk * 16, 16)] = v & jnp.int32(0xFFFF)
        didx2_v[b, pl.ds(kk * 16, 16)] = lax.shift_right_logical(v, 16)


@functools.partial(
    pl.kernel,
    mesh=_mesh,
    out_type=jax.ShapeDtypeStruct((NCORE, ACC_ROWS, FF), jnp.float32),
    scratch_types=[
        pltpu.VMEM((CCH, KCH), jnp.int32),
        pltpu.VMEM((KCH, FF), jnp.float32),
        pltpu.VMEM_SHARED((ACC_ROWS, FF), jnp.float32),
    ],
)
def _sc_deg(pk_hbm, zeros_hbm, ones_hbm, out_hbm, pidx_v, ones_v, deg_sh):
    c = lax.axis_index("c")
    s = lax.axis_index("s")
    w = s * NCORE + c
    _zero_acc(zeros_hbm, deg_sh, s)
    pltpu.sync_copy(pk_hbm.at[pl.ds(w * CCH, CCH)], pidx_v)
    pltpu.sync_copy(ones_hbm, ones_v)

    def unp(r, carry):
        for kk in range(KCH // 16):
            v = pidx_v[r, pl.ds(kk * 16, 16)]
            pidx_v[r, pl.ds(kk * 16, 16)] = lax.shift_right_logical(v, 16)
        return carry

    lax.fori_loop(0, CCH, unp, 0)
    plsc.subcore_barrier()

    def body(i, carry):
        pltpu.sync_copy(ones_v, deg_sh.at[pidx_v.at[i]], add=True)
        return carry

    lax.fori_loop(0, CCH, body, 0)
    plsc.subcore_barrier()
    _copy_out(deg_sh, out_hbm, c, s)


@functools.partial(
    pl.kernel,
    mesh=_mesh,
    out_type=jax.ShapeDtypeStruct((NCORE, ACC_ROWS, FF), jnp.float32),
    scratch_types=[
        pltpu.VMEM((3, 1, KCH), jnp.int32),
        pltpu.VMEM((3, KCH), jnp.int32),
        pltpu.VMEM((3, KCH), jnp.int32),
        pltpu.VMEM((KCH, FF), jnp.float32),
        pltpu.VMEM((KCH, FF), jnp.float32),
        pltpu.VMEM((KCH, FF), jnp.float32),
        pltpu.VMEM_SHARED((ACC_ROWS, FF), jnp.float32),
        pltpu.SemaphoreType.DMA,
        pltpu.SemaphoreType.DMA,
        pltpu.SemaphoreType.DMA,
        pltpu.SemaphoreType.DMA,
        pltpu.SemaphoreType.DMA,
        pltpu.SemaphoreType.DMA,
    ],
)
def _sc_spmm(g_hbm, pk_hbm, zeros_hbm, out_hbm,
             pk2_v, sidx2_v, didx2_v, rows0_v, rows1_v, rows2_v, acc_sh,
             sg0, sg1, sg2, ss0, ss1, ss2):
    c = lax.axis_index("c")
    s = lax.axis_index("s")
    bufs = (rows0_v, rows1_v, rows2_v)
    gsems = (sg0, sg1, sg2)
    ssems = (ss0, ss1, ss2)
    with jax.named_scope("ph_zero"):
        _zero_acc(zeros_hbm, acc_sh, s)

    def fetch(off, q, b):
        pltpu.sync_copy(pk_hbm.at[pl.ds(off + q, 1)], pk2_v.at[pl.ds(b, 1)])
        _unpack_chunk(pk2_v, sidx2_v, didx2_v, b)

    def pre(off):
        fetch(off, 0, 0)
        pltpu.async_copy(g_hbm.at[sidx2_v.at[0]], bufs[0], gsems[0])
        fetch(off, 1, 1)
        pltpu.async_copy(g_hbm.at[sidx2_v.at[1]], bufs[1], gsems[1])

    def run(nch, off):
        def body(i, carry):
            for u in range(3):
                q = i * 3 + u
                b = u
                b2 = (u + 2) % 3

                @pl.when((q >= 1) & (q <= nch))
                def _():
                    pltpu.make_async_copy(g_hbm.at[pl.ds(0, KCH)],
                                          bufs[b2], ssems[b2]).wait()

                @pl.when(q + 2 < nch)
                def _():
                    fetch(off, q + 2, b2)
                    pltpu.async_copy(g_hbm.at[sidx2_v.at[b2]], bufs[b2],
                                     gsems[b2])

                @pl.when(q < nch)
                def _():
                    pltpu.make_async_copy(g_hbm.at[pl.ds(0, KCH)], bufs[b],
                                          gsems[b]).wait()
                    pltpu.async_copy(bufs[b], acc_sh.at[didx2_v.at[b]],
                                     ssems[b], add=True)
            return carry

        lax.fori_loop(0, (nch + 2) // 3, body, 0)
        if nch % 3 == 0:
            pltpu.make_async_copy(g_hbm.at[pl.ds(0, KCH)],
                                  bufs[(nch - 1) % 3],
                                  ssems[(nch - 1) % 3]).wait()

    with jax.named_scope("ph_pre"):
        @pl.when(c == 0)
        def _():
            pre(s * CA)

        if CB:
            @pl.when(c == 1)
            def _():
                pre(NSUB * CA + s * CB)

        plsc.subcore_barrier()

    with jax.named_scope("ph_loop"):
        @pl.when(c == 0)
        def _():
            run(CA, s * CA)

        if CB:
            @pl.when(c == 1)
            def _():
                run(CB, NSUB * CA + s * CB)

        plsc.subcore_barrier()

    with jax.named_scope("ph_out"):
        _copy_out(acc_sh, out_hbm, c, s)


def _prep_body(parts_ref, dinv_ref):
    deg = parts_ref[0, :NN, :1] + parts_ref[1, :NN, :1] + 1.0
    dinv_ref[...] = lax.rsqrt(deg)


def _tc_prep(deg_parts):
    return pl.pallas_call(
        _prep_body,
        out_shape=jax.ShapeDtypeStruct((NN, 1), jnp.float32),
    )(deg_parts)


def _mm1_body(x_ref, w_ref, dinv_ref, hlin_ref, g_ref):
    hl = jnp.dot(x_ref[...], w_ref[...],
                 preferred_element_type=jnp.float32,
                 precision=lax.Precision.HIGHEST)
    hlin_ref[...] = hl
    g_ref[...] = hl * dinv_ref[...]


def _tc_mm1(x, w, dinv):
    return pl.pallas_call(
        _mm1_body,
        grid=(NN // BLK,),
        in_specs=[
            pl.BlockSpec((BLK, FF), lambda i: (i, 0)),
            pl.BlockSpec((FF, FF), lambda i: (0, 0)),
            pl.BlockSpec((BLK, 1), lambda i: (i, 0)),
        ],
        out_specs=[
            pl.BlockSpec((BLK, FF), lambda i: (i, 0)),
            pl.BlockSpec((BLK, FF), lambda i: (i, 0)),
        ],
        out_shape=[
            jax.ShapeDtypeStruct((NN, FF), jnp.float32),
            jax.ShapeDtypeStruct((NN, FF), jnp.float32),
        ],
    )(x, w, dinv)


def _mid_body(acc_ref, hlin_ref, dinv_ref, b_ref, w_ref, hlinn_ref, gn_ref):
    dv = dinv_ref[...]
    a = acc_ref[0] + acc_ref[1]
    sfull = dv * a + (dv * dv) * hlin_ref[...] + b_ref[...]
    h = jnp.maximum(sfull, 0.0)
    hn = jnp.dot(h, w_ref[...],
                 preferred_element_type=jnp.float32,
                 precision=lax.Precision.HIGHEST)
    hlinn_ref[...] = hn
    gn_ref[...] = hn * dv


def _tc_mid(acc, hlin, dinv, b, w):
    return pl.pallas_call(
        _mid_body,
        grid=(NN // BLK,),
        in_specs=[
            pl.BlockSpec((NCORE, BLK, FF), lambda i: (0, i, 0)),
            pl.BlockSpec((BLK, FF), lambda i: (i, 0)),
            pl.BlockSpec((BLK, 1), lambda i: (i, 0)),
            pl.BlockSpec((1, FF), lambda i: (0, 0)),
            pl.BlockSpec((FF, FF), lambda i: (0, 0)),
        ],
        out_specs=[
            pl.BlockSpec((BLK, FF), lambda i: (i, 0)),
            pl.BlockSpec((BLK, FF), lambda i: (i, 0)),
        ],
        out_shape=[
            jax.ShapeDtypeStruct((NN, FF), jnp.float32),
            jax.ShapeDtypeStruct((NN, FF), jnp.float32),
        ],
    )(acc, hlin, dinv, b, w)


def _final_body(acc_ref, hlin_ref, dinv_ref, b_ref, out_ref):
    dv = dinv_ref[...]
    a = acc_ref[0] + acc_ref[1]
    out_ref[...] = dv * a + (dv * dv) * hlin_ref[...] + b_ref[...]


def _tc_final(acc, hlin, dinv, b):
    return pl.pallas_call(
        _final_body,
        grid=(NN // BLK,),
        in_specs=[
            pl.BlockSpec((NCORE, BLK, FF), lambda i: (0, i, 0)),
            pl.BlockSpec((BLK, FF), lambda i: (i, 0)),
            pl.BlockSpec((BLK, 1), lambda i: (i, 0)),
            pl.BlockSpec((1, FF), lambda i: (0, 0)),
        ],
        out_specs=pl.BlockSpec((BLK, FF), lambda i: (i, 0)),
        out_shape=jax.ShapeDtypeStruct((NN, FF), jnp.float32),
    )(acc, hlin, dinv, b)


def kernel(x, edge_index, W1, b1, W2, b2, W3, b3):
    src = edge_index[0]
    dst = edge_index[1]
    pad = EPAD - EE
    packed = jnp.left_shift(dst, 16) | src
    pkflat = jnp.concatenate(
        [packed, jnp.full((pad,), NN << 16, jnp.int32)])
    pk2d = pkflat.reshape(NCHUNK, KCH)
    pk3d = pkflat.reshape(NCHUNK, 1, KCH)

    zeros_rows = jnp.zeros((KCH, FF), jnp.float32)
    ones_rows = jnp.ones((KCH, FF), jnp.float32)

    deg_parts = _sc_deg(pk2d, zeros_rows, ones_rows)
    dinv = _tc_prep(deg_parts)

    hlin1, g1 = _tc_mm1(x, W1, dinv)
    acc1 = _sc_spmm(g1, pk3d, zeros_rows)
    hlin2, g2 = _tc_mid(acc1, hlin1, dinv, b1.reshape(1, FF), W2)
    acc2 = _sc_spmm(g2, pk3d, zeros_rows)
    hlin3, g3 = _tc_mid(acc2, hlin2, dinv, b2.reshape(1, FF), W3)
    acc3 = _sc_spmm(g3, pk3d, zeros_rows)
    return _tc_final(acc3, hlin3, dinv, b3.reshape(1, FF))
